# baseline (speedup 1.0000x reference)
import math

import jax
import jax.numpy as jnp
from jax.experimental import pallas as pl
from jax.experimental.pallas import tpu as pltpu


def _attn_kernel(xq_ref, k_ref, v_ref, ks_ref, vs_ref, mask_ref, o_ref):
    G = k_ref.shape[1]
    D = xq_ref.shape[-1]
    inv_sqrt_d = 1.0 / math.sqrt(D)
    scale_row = ks_ref[0] * inv_sqrt_d
    mask_row = mask_ref[0]
    vs_row = vs_ref[0]
    for g in range(G):
        q = xq_ref[0, g]
        k = k_ref[0, g].astype(jnp.float32)
        scores = jax.lax.dot_general(
            q, k, (((1,), (1,)), ((), ())),
            preferred_element_type=jnp.float32)
        scores = scores * scale_row + mask_row
        m = jnp.max(scores, axis=-1, keepdims=True)
        e = jnp.exp(scores - m)
        s = jnp.sum(e, axis=-1, keepdims=True)
        p = e * vs_row
        v = v_ref[0, g].astype(jnp.float32)
        acc = jax.lax.dot_general(
            p, v, (((1,), (0,)), ((), ())),
            preferred_element_type=jnp.float32)
        o_ref[0, g] = acc / s


def kernel(xq, keys, values, k_scaler, v_scaler, mask):
    B, H, _, D = xq.shape
    Hkv, S = keys.shape[1], keys.shape[2]
    n_rep = H // Hkv
    xqg = xq.reshape(B, Hkv, n_rep, D)
    ks = k_scaler.reshape(B, 1, S)
    vs = v_scaler.reshape(B, 1, S)
    msk = mask.reshape(B, 1, S)
    G = 4
    out = pl.pallas_call(
        _attn_kernel,
        grid=(B, Hkv // G),
        in_specs=[
            pl.BlockSpec((1, G, n_rep, D), lambda b, j: (b, j, 0, 0)),
            pl.BlockSpec((1, G, S, D), lambda b, j: (b, j, 0, 0)),
            pl.BlockSpec((1, G, S, D), lambda b, j: (b, j, 0, 0)),
            pl.BlockSpec((1, 1, S), lambda b, j: (b, 0, 0)),
            pl.BlockSpec((1, 1, S), lambda b, j: (b, 0, 0)),
            pl.BlockSpec((1, 1, S), lambda b, j: (b, 0, 0)),
        ],
        out_specs=pl.BlockSpec((1, G, n_rep, D), lambda b, j: (b, j, 0, 0)),
        out_shape=jax.ShapeDtypeStruct((B, Hkv, n_rep, D), jnp.float32),
        compiler_params=pltpu.CompilerParams(
            dimension_semantics=("parallel", "parallel"),
        ),
        name="int8_kv_decode_attn",
    )(xqg, keys, values, ks, vs, msk)
    return out.reshape(B, H, 1, D)


# speedup vs baseline: 1.0022x; 1.0022x over previous
import math

import jax
import jax.numpy as jnp
from jax.experimental import pallas as pl
from jax.experimental.pallas import tpu as pltpu

_NC = 2


def _attn_kernel(*refs):
    xq_ref = refs[0]
    k_refs = refs[1:1 + _NC]
    v_refs = refs[1 + _NC:1 + 2 * _NC]
    ks_ref, vs_ref, mask_ref, o_ref = refs[1 + 2 * _NC:]
    G = k_refs[0].shape[1]
    S2 = k_refs[0].shape[3]
    D = xq_ref.shape[-1]
    inv_sqrt_d = 1.0 / math.sqrt(D)
    scale_row = ks_ref[0] * inv_sqrt_d
    mask_row = mask_ref[0]
    vs_row = vs_ref[0]
    for g in range(G):
        q = xq_ref[0, g]
        parts = [
            jax.lax.dot_general(
                q, kc[0, g, 0].astype(jnp.float32), (((1,), (1,)), ((), ())),
                preferred_element_type=jnp.float32)
            for kc in k_refs
        ]
        scores = jnp.concatenate(parts, axis=-1)
        scores = scores * scale_row + mask_row
        m = jnp.max(scores, axis=-1, keepdims=True)
        e = jnp.exp(scores - m)
        s = jnp.sum(e, axis=-1, keepdims=True)
        p = e * vs_row
        acc = 0.0
        for c, vc in enumerate(v_refs):
            acc = acc + jax.lax.dot_general(
                p[:, c * S2:(c + 1) * S2], vc[0, g, 0].astype(jnp.float32),
                (((1,), (0,)), ((), ())),
                preferred_element_type=jnp.float32)
        o_ref[0, g] = acc / s


def kernel(xq, keys, values, k_scaler, v_scaler, mask):
    B, H, _, D = xq.shape
    Hkv, S = keys.shape[1], keys.shape[2]
    n_rep = H // Hkv
    xqg = xq.reshape(B, Hkv, n_rep, D)
    karr = keys.reshape(B, Hkv, _NC, S // _NC, D)
    varr = values.reshape(B, Hkv, _NC, S // _NC, D)
    ks = k_scaler.reshape(B, 1, S)
    vs = v_scaler.reshape(B, 1, S)
    msk = mask.reshape(B, 1, S)
    G = Hkv

    def chunk_spec(c):
        return pl.BlockSpec((1, G, 1, S // _NC, D),
                            lambda b, j, c=c: (b, j, c, 0, 0))

    in_specs = (
        [pl.BlockSpec((1, G, n_rep, D), lambda b, j: (b, j, 0, 0))]
        + [chunk_spec(c) for c in range(_NC)]
        + [chunk_spec(c) for c in range(_NC)]
        + [pl.BlockSpec((1, 1, S), lambda b, j: (b, 0, 0))] * 3
    )
    out = pl.pallas_call(
        _attn_kernel,
        grid=(B, Hkv // G),
        in_specs=in_specs,
        out_specs=pl.BlockSpec((1, G, n_rep, D), lambda b, j: (b, j, 0, 0)),
        out_shape=jax.ShapeDtypeStruct((B, Hkv, n_rep, D), jnp.float32),
        compiler_params=pltpu.CompilerParams(
            dimension_semantics=("parallel", "parallel"),
        ),
        name="int8_kv_decode_attn",
    )(xqg, *([karr] * _NC), *([varr] * _NC), ks, vs, msk)
    return out.reshape(B, H, 1, D)


# speedup vs baseline: 1.3920x; 1.3890x over previous
import math

import jax
import jax.numpy as jnp
from jax.experimental import pallas as pl
from jax.experimental.pallas import tpu as pltpu


def _attn_kernel(xq_ref, k_ref, v_ref, ks_ref, vs_ref, mask_ref, o_ref):
    G = k_ref.shape[1]
    D = xq_ref.shape[-1]
    inv_sqrt_d = 1.0 / math.sqrt(D)
    scale_row = ks_ref[0] * inv_sqrt_d
    mask_row = mask_ref[0]
    vs_row = vs_ref[0]

    def qk(g):
        q = xq_ref[0, g].astype(jnp.bfloat16)
        k = k_ref[0, g].astype(jnp.bfloat16)
        scores = jax.lax.dot_general(
            q, k, (((1,), (1,)), ((), ())),
            preferred_element_type=jnp.float32)
        return scores * scale_row + mask_row

    def finish(g, scores):
        m = jnp.max(scores, axis=-1, keepdims=True)
        e = jnp.exp(scores - m)
        s = jnp.sum(e, axis=-1, keepdims=True)
        p = (e * vs_row).astype(jnp.bfloat16)
        v = v_ref[0, g].astype(jnp.bfloat16)
        acc = jax.lax.dot_general(
            p, v, (((1,), (0,)), ((), ())),
            preferred_element_type=jnp.float32)
        o_ref[0, g] = acc / s

    pending = [qk(0), qk(1)]
    for g in range(2, G):
        pending.append(qk(g))
        finish(g - 2, pending.pop(0))
    finish(G - 2, pending[0])
    finish(G - 1, pending[1])


def kernel(xq, keys, values, k_scaler, v_scaler, mask):
    B, H, _, D = xq.shape
    Hkv, S = keys.shape[1], keys.shape[2]
    n_rep = H // Hkv
    xqg = xq.reshape(B, Hkv, n_rep, D)
    ks = k_scaler.reshape(B, 1, S)
    vs = v_scaler.reshape(B, 1, S)
    msk = mask.reshape(B, 1, S)
    G = Hkv
    out = pl.pallas_call(
        _attn_kernel,
        grid=(B, Hkv // G),
        in_specs=[
            pl.BlockSpec((1, G, n_rep, D), lambda b, j: (b, j, 0, 0)),
            pl.BlockSpec((1, G, S, D), lambda b, j: (b, j, 0, 0)),
            pl.BlockSpec((1, G, S, D), lambda b, j: (b, j, 0, 0)),
            pl.BlockSpec((1, 1, S), lambda b, j: (b, 0, 0)),
            pl.BlockSpec((1, 1, S), lambda b, j: (b, 0, 0)),
            pl.BlockSpec((1, 1, S), lambda b, j: (b, 0, 0)),
        ],
        out_specs=pl.BlockSpec((1, G, n_rep, D), lambda b, j: (b, j, 0, 0)),
        out_shape=jax.ShapeDtypeStruct((B, Hkv, n_rep, D), jnp.float32),
        compiler_params=pltpu.CompilerParams(
            dimension_semantics=("parallel", "parallel"),
        ),
        name="int8_kv_decode_attn",
    )(xqg, keys, values, ks, vs, msk)
    return out.reshape(B, H, 1, D)
